# baseline (speedup 1.0000x reference)
import functools

import jax
import jax.numpy as jnp
from jax.experimental import pallas as pl
from jax.experimental.pallas import tpu as pltpu

_VMEM_LIMIT = 48 * 1024 * 1024


def _sigmoid(x):
    return pl.reciprocal(1.0 + jnp.exp(-x), approx=True)


def _one_batch(x_ref, w_ref, b_ref, xw_ref, sy_ref, selh_ref, selht_ref,
               wm_ref, o_ref, g, inv_m, inv_c, h, w):
    x = x_ref[g]
    m, c = x.shape

    ones_m = jnp.full((8, m), inv_m, jnp.float32)
    mean8 = jnp.dot(ones_m, x, preferred_element_type=jnp.float32)
    z8 = jnp.dot(mean8, w_ref[...], preferred_element_type=jnp.float32)
    ca = _sigmoid(z8 + b_ref[...])[0:1]

    y = x * ca

    ones_c = jnp.full((c, w), inv_c, jnp.bfloat16)
    meanm = jnp.dot(y.astype(jnp.bfloat16), ones_c,
                    preferred_element_type=jnp.float32)
    maxm = jnp.broadcast_to(jnp.max(y, axis=1, keepdims=True), (m, w))
    yb = jnp.concatenate([meanm, maxm], axis=1) * wm_ref[...]

    zz = jnp.dot(selh_ref[...], yb, preferred_element_type=jnp.float32)
    s1 = jnp.dot(zz, xw_ref[...], preferred_element_type=jnp.float32)
    parts = [
        jnp.dot(sy_ref[i * h:(i + 1) * h, :], s1[:, i * w:(i + 1) * w],
                preferred_element_type=jnp.float32)
        for i in range(7)
    ]
    conv = ((parts[0] + parts[1]) + (parts[2] + parts[3])) + (
        (parts[4] + parts[5]) + parts[6])
    sa2 = _sigmoid(conv)

    tmp = jnp.dot(selht_ref[...], sa2.astype(jnp.bfloat16),
                  preferred_element_type=jnp.float32)
    sacol = jnp.sum(tmp * wm_ref[:, 0:w], axis=1, keepdims=True)

    o_ref[g] = y * sacol


def _cbam_kernel(x_ref, w_ref, b_ref, xw_ref, sy_ref, selh_ref, selht_ref,
                 wm_ref, o_ref, *, inv_m, inv_c, h, w, gsz):
    for g in range(gsz):
        _one_batch(x_ref, w_ref, b_ref, xw_ref, sy_ref, selh_ref,
                   selht_ref, wm_ref, o_ref, g, inv_m, inv_c, h, w)


def _build_consts(w_sa, h, w):
    f32 = jnp.float32
    w3 = w_sa.reshape(2, 7, 7)
    pw = jnp.arange(w, dtype=jnp.int32)
    dx = pw[:, None] - pw[None, :] + 3
    band = (dx[None] == jnp.arange(7, dtype=jnp.int32)[:, None, None])
    bandf = band.astype(f32)
    xw = jnp.einsum('kij,jab->kiab', w3, bandf)
    xwcat = xw.transpose(0, 2, 1, 3).reshape(2 * w, 7 * w)

    hy = jnp.arange(h, dtype=jnp.int32)
    dy = hy[:, None] - hy[None, :] + 3
    sy = (dy[None] == jnp.arange(7, dtype=jnp.int32)[:, None, None])
    syf = sy.transpose(0, 2, 1).astype(f32).reshape(7 * h, h)

    p = jnp.arange(h * w, dtype=jnp.int32)
    selht = (p[:, None] // w == hy[None, :]).astype(jnp.bfloat16)
    selh = selht.T.astype(f32)
    w2 = jnp.arange(2 * w, dtype=jnp.int32)
    wm = (p[:, None] % w == w2[None, :] % w).astype(f32)
    return xwcat, syf, selh, selht, wm


def kernel(x, w_fc, b_fc, w_sa):
    B, C, H, W = x.shape
    M = H * W
    xh = jnp.transpose(x, (0, 2, 3, 1)).reshape(B, M, C)
    xwcat, syf, selh, selht, wm = _build_consts(w_sa, H, W)

    G = 4
    while G > 1 and B % (2 * G):
        G //= 2
    body = functools.partial(
        _cbam_kernel, inv_m=1.0 / float(M), inv_c=1.0 / float(C), h=H, w=W,
        gsz=G)
    nb = B // (2 * G)
    cspec = lambda shape: pl.BlockSpec(shape, lambda ci, t: tuple(0 for _ in shape))
    outh = pl.pallas_call(
        body,
        out_shape=jax.ShapeDtypeStruct((B, M, C), x.dtype),
        grid=(2, nb),
        in_specs=[
            pl.BlockSpec((G, M, C), lambda ci, t: (ci * nb + t, 0, 0)),
            cspec((C, C)),
            cspec((1, C)),
            cspec((2 * W, 7 * W)),
            cspec((7 * H, H)),
            cspec((H, M)),
            cspec((M, H)),
            cspec((M, 2 * W)),
        ],
        out_specs=pl.BlockSpec((G, M, C), lambda ci, t: (ci * nb + t, 0, 0)),
        compiler_params=pltpu.CompilerParams(
            dimension_semantics=("parallel", "arbitrary"),
            vmem_limit_bytes=_VMEM_LIMIT),
    )(xh, w_fc, b_fc, xwcat, syf, selh, selht, wm)
    return jnp.transpose(outh.reshape(B, H, W, C), (0, 3, 1, 2))


# speedup vs baseline: 1.0091x; 1.0091x over previous
import functools

import jax
import jax.numpy as jnp
from jax.experimental import pallas as pl
from jax.experimental.pallas import tpu as pltpu

_VMEM_LIMIT = 48 * 1024 * 1024


def _sigmoid(x):
    return pl.reciprocal(1.0 + jnp.exp(-x), approx=True)


def _one_batch(x_ref, w_ref, b_ref, xw_ref, sy_ref, selh_ref, selht_ref,
               wm_ref, o_ref, g, inv_m, inv_c, h, w):
    x = x_ref[g]
    m, c = x.shape

    ones_m = jnp.full((8, m), inv_m, jnp.float32)
    mean8 = jnp.dot(ones_m, x, preferred_element_type=jnp.float32)
    z8 = jnp.dot(mean8, w_ref[...], preferred_element_type=jnp.float32)
    ca = _sigmoid(z8 + b_ref[...])[0:1]

    y = x * ca

    ones_c = jnp.full((c, w), inv_c, jnp.bfloat16)
    meanm = jnp.dot(y.astype(jnp.bfloat16), ones_c,
                    preferred_element_type=jnp.float32)
    maxm = jnp.broadcast_to(jnp.max(y, axis=1, keepdims=True), (m, w))
    yb = jnp.concatenate([meanm, maxm], axis=1) * wm_ref[...]

    zz = jnp.dot(selh_ref[...], yb, preferred_element_type=jnp.float32)
    s1 = jnp.dot(zz, xw_ref[...], preferred_element_type=jnp.float32)
    parts = [
        jnp.dot(sy_ref[i * h:(i + 1) * h, :], s1[:, i * w:(i + 1) * w],
                preferred_element_type=jnp.float32)
        for i in range(7)
    ]
    conv = ((parts[0] + parts[1]) + (parts[2] + parts[3])) + (
        (parts[4] + parts[5]) + parts[6])
    sa2 = _sigmoid(conv)

    tmp = jnp.dot(selht_ref[...], sa2.astype(jnp.bfloat16),
                  preferred_element_type=jnp.float32)
    sacol = jnp.sum(tmp * wm_ref[:, 0:w], axis=1, keepdims=True)

    o_ref[g] = y * sacol


def _cbam_kernel(x_ref, w_ref, b_ref, xw_ref, sy_ref, selh_ref, selht_ref,
                 wm_ref, o_ref, *, inv_m, inv_c, h, w, gsz):
    for g in range(gsz):
        _one_batch(x_ref, w_ref, b_ref, xw_ref, sy_ref, selh_ref,
                   selht_ref, wm_ref, o_ref, g, inv_m, inv_c, h, w)


def _build_consts(w_sa, h, w):
    f32 = jnp.float32
    w3 = w_sa.reshape(2, 7, 7)
    pw = jnp.arange(w, dtype=jnp.int32)
    dx = pw[:, None] - pw[None, :] + 3
    band = (dx[None] == jnp.arange(7, dtype=jnp.int32)[:, None, None])
    bandf = band.astype(f32)
    xw = jnp.einsum('kij,jab->kiab', w3, bandf)
    xwcat = xw.transpose(0, 2, 1, 3).reshape(2 * w, 7 * w)

    hy = jnp.arange(h, dtype=jnp.int32)
    dy = hy[:, None] - hy[None, :] + 3
    sy = (dy[None] == jnp.arange(7, dtype=jnp.int32)[:, None, None])
    syf = sy.transpose(0, 2, 1).astype(f32).reshape(7 * h, h)

    p = jnp.arange(h * w, dtype=jnp.int32)
    selht = (p[:, None] // w == hy[None, :]).astype(jnp.bfloat16)
    selh = selht.T.astype(f32)
    w2 = jnp.arange(2 * w, dtype=jnp.int32)
    wm = (p[:, None] % w == w2[None, :] % w).astype(f32)
    return xwcat, syf, selh, selht, wm


def kernel(x, w_fc, b_fc, w_sa):
    B, C, H, W = x.shape
    M = H * W
    xh = jnp.transpose(x, (0, 2, 3, 1)).reshape(B, M, C)
    xwcat, syf, selh, selht, wm = _build_consts(w_sa, H, W)

    G = 8
    while G > 1 and B % (2 * G):
        G //= 2
    body = functools.partial(
        _cbam_kernel, inv_m=1.0 / float(M), inv_c=1.0 / float(C), h=H, w=W,
        gsz=G)
    nb = B // (2 * G)
    cspec = lambda shape: pl.BlockSpec(shape, lambda ci, t: tuple(0 for _ in shape))
    outh = pl.pallas_call(
        body,
        out_shape=jax.ShapeDtypeStruct((B, M, C), x.dtype),
        grid=(2, nb),
        in_specs=[
            pl.BlockSpec((G, M, C), lambda ci, t: (ci * nb + t, 0, 0)),
            cspec((C, C)),
            cspec((1, C)),
            cspec((2 * W, 7 * W)),
            cspec((7 * H, H)),
            cspec((H, M)),
            cspec((M, H)),
            cspec((M, 2 * W)),
        ],
        out_specs=pl.BlockSpec((G, M, C), lambda ci, t: (ci * nb + t, 0, 0)),
        compiler_params=pltpu.CompilerParams(
            dimension_semantics=("parallel", "arbitrary"),
            vmem_limit_bytes=_VMEM_LIMIT),
    )(xh, w_fc, b_fc, xwcat, syf, selh, selht, wm)
    return jnp.transpose(outh.reshape(B, H, W, C), (0, 3, 1, 2))
